# baseline (speedup 1.0000x reference)
import dataclasses
import functools

import jax
import jax.numpy as jnp
from jax import lax
from jax.experimental import pallas as pl
from jax.experimental.pallas import tpu as pltpu
from jax.experimental.pallas import tpu_sc as plsc

N = 10000
NP = 10240
E = 320000
H = 64
NC = 2
NS = 16
EPT = E // (NC * NS)
CH = 80
NCH = EPT // CH
RPT = NP // NS

BLK = 1024
GRID = NP // BLK

_SELU_L = 1.0507009873554805
_SELU_A = 1.6732632423543772


def _selu(x):
    return _SELU_L * jnp.where(x > 0, x, _SELU_A * (jnp.exp(x) - 1.0))


def _proj_body(x_ref, w_ref, al_ref, ar_ref, h_ref, el_ref, er_ref, mel_ref, mer_ref):
    i = pl.program_id(0)
    h = jnp.dot(x_ref[...], w_ref[...], preferred_element_type=jnp.float32)
    h_ref[...] = h
    el = jnp.sum(h * al_ref[...][None, :], axis=1)
    er = jnp.sum(h * ar_ref[...][None, :], axis=1)
    el_ref[...] = jnp.broadcast_to(el[None, :], (8, BLK))
    er_ref[...] = jnp.broadcast_to(er[None, :], (8, BLK))
    ml = jnp.full((8, 128), jnp.max(el), dtype=jnp.float32)
    mr = jnp.full((8, 128), jnp.max(er), dtype=jnp.float32)

    @pl.when(i == 0)
    def _():
        mel_ref[...] = ml
        mer_ref[...] = mr

    @pl.when(i > 0)
    def _():
        mel_ref[...] = jnp.maximum(mel_ref[...], ml)
        mer_ref[...] = jnp.maximum(mer_ref[...], mr)


def _project(x, w, al, ar):
    d = x.shape[1]
    f32 = jnp.float32
    return pl.pallas_call(
        _proj_body,
        grid=(GRID,),
        in_specs=[
            pl.BlockSpec((BLK, d), lambda i: (i, 0)),
            pl.BlockSpec((d, H), lambda i: (0, 0)),
            pl.BlockSpec((H,), lambda i: (0,)),
            pl.BlockSpec((H,), lambda i: (0,)),
        ],
        out_specs=[
            pl.BlockSpec((BLK, H), lambda i: (i, 0)),
            pl.BlockSpec((8, BLK), lambda i: (0, i)),
            pl.BlockSpec((8, BLK), lambda i: (0, i)),
            pl.BlockSpec((8, 128), lambda i: (0, 0)),
            pl.BlockSpec((8, 128), lambda i: (0, 0)),
        ],
        out_shape=[
            jax.ShapeDtypeStruct((NP, H), f32),
            jax.ShapeDtypeStruct((8, NP), f32),
            jax.ShapeDtypeStruct((8, NP), f32),
            jax.ShapeDtypeStruct((8, 128), f32),
            jax.ShapeDtypeStruct((8, 128), f32),
        ],
    )(x, w, al, ar)


def _merge_body(wp_ref, dp_ref, b_ref, h_ref):
    w = wp_ref[0] + wp_ref[1]
    dn = dp_ref[0] + dp_ref[1]
    h_ref[...] = _selu(w / (dn[:, None] + 1e-9) + b_ref[...][None, :])


def _merge(wparts, dparts, b):
    return pl.pallas_call(
        _merge_body,
        grid=(GRID,),
        in_specs=[
            pl.BlockSpec((2, BLK, H), lambda i: (0, i, 0)),
            pl.BlockSpec((2, BLK), lambda i: (0, i)),
            pl.BlockSpec((H,), lambda i: (0,)),
        ],
        out_specs=pl.BlockSpec((BLK, H), lambda i: (i, 0)),
        out_shape=jax.ShapeDtypeStruct((NP, H), jnp.float32),
    )(wparts, dparts, b)


def _head_body(wp_ref, dp_ref, b_ref, f1w_ref, f1b_ref, f2w_ref, f2b_ref,
               o_ref, gsum_ref):
    i = pl.program_id(0)
    w = wp_ref[0] + wp_ref[1]
    dn = dp_ref[0] + dp_ref[1]
    h3 = _selu(w / (dn[:, None] + 1e-9) + b_ref[...][None, :])
    row = lax.broadcasted_iota(jnp.int32, (BLK, H), 0) + i * BLK
    h3 = jnp.where(row < N, h3, 0.0)
    s = jnp.broadcast_to(jnp.sum(h3, axis=0)[None, :], (8, H))

    @pl.when(i == 0)
    def _():
        gsum_ref[...] = s

    @pl.when(i > 0)
    def _():
        gsum_ref[...] = gsum_ref[...] + s

    @pl.when(i == GRID - 1)
    def _():
        g = gsum_ref[0:1, :] * (1.0 / N)
        z = _selu(jnp.dot(g, f1w_ref[...], preferred_element_type=jnp.float32)
                  + f1b_ref[...][None, :])
        z = (jnp.dot(z, f2w_ref[...], preferred_element_type=jnp.float32)
             + f2b_ref[...][None, :])
        m = jnp.max(z, axis=1, keepdims=True)
        lse = m + jnp.log(jnp.sum(jnp.exp(z - m), axis=1, keepdims=True))
        o_ref[...] = z - lse


def _head(wparts, dparts, b, f1w, f1b, f2w, f2b):
    h2 = f1w.shape[1]
    c = f2w.shape[1]
    return pl.pallas_call(
        _head_body,
        grid=(GRID,),
        in_specs=[
            pl.BlockSpec((2, BLK, H), lambda i: (0, i, 0)),
            pl.BlockSpec((2, BLK), lambda i: (0, i)),
            pl.BlockSpec((H,), lambda i: (0,)),
            pl.BlockSpec((H, h2), lambda i: (0, 0)),
            pl.BlockSpec((h2,), lambda i: (0,)),
            pl.BlockSpec((h2, c), lambda i: (0, 0)),
            pl.BlockSpec((c,), lambda i: (0,)),
        ],
        out_specs=pl.BlockSpec((1, c), lambda i: (0, 0)),
        out_shape=jax.ShapeDtypeStruct((1, c), jnp.float32),
        scratch_shapes=[pltpu.VMEM((8, H), jnp.float32)],
    )(wparts, dparts, b, f1w, f1b, f2w, f2b)


def _edge_pass(h, el8, er8, mel, mer, edge_index):
    f32 = jnp.float32
    mesh = plsc.VectorSubcoreMesh(core_axis_name="c", subcore_axis_name="s")
    cp = pltpu.CompilerParams()
    if "needs_layout_passes" in pltpu.CompilerParams.__dataclass_fields__:
        cp = dataclasses.replace(cp, needs_layout_passes=False)
    if "use_tc_tiling_on_sc" in pltpu.CompilerParams.__dataclass_fields__:
        cp = dataclasses.replace(cp, use_tc_tiling_on_sc=False)

    @functools.partial(
        pl.kernel,
        compiler_params=cp,
        out_type=[
            jax.ShapeDtypeStruct((NC, NP, H), f32),
            jax.ShapeDtypeStruct((NC, NP), f32),
        ],
        mesh=mesh,
        scratch_types=(
            [
                pltpu.VMEM((NP,), f32),
                pltpu.VMEM((NP,), f32),
                pltpu.VMEM((16,), f32),
                pltpu.VMEM((16,), f32),
                pltpu.VMEM((NCH, CH), jnp.int32),
                pltpu.VMEM((NCH, CH), jnp.int32),
            ]
            + [pltpu.VMEM((CH, H), f32)] * 3
            + [pltpu.VMEM((CH,), f32)] * 3
            + [pltpu.SemaphoreType.DMA] * 9
            + [
                pltpu.VMEM_SHARED((NP, H), f32),
                pltpu.VMEM_SHARED((NP,), f32),
            ]
        ),
    )
    def k(h_hbm, el_hbm, er_hbm, mel_hbm, mer_hbm, ei_hbm, wout_hbm, dout_hbm,
          *scr):
        el_v, er_v, gm_v, gm2_v, src_a, dst_a = scr[0:6]
        rows, eeb = scr[6:9], scr[9:12]
        sg, sr, se = scr[12:15], scr[15:18], scr[18:21]
        w_sh, d_sh = scr[21], scr[22]

        c = lax.axis_index("c")
        s = lax.axis_index("s")
        wid = c * NS + s

        def issue_gather(ci, b):
            pltpu.async_copy(h_hbm.at[src_a.at[ci]], rows[b], sg[b])

        def wait_gather(b):
            pltpu.make_async_copy(h_hbm.at[src_a.at[0]], rows[b],
                                  sg[b]).wait()

        def issue_scatter(ci, b):
            pltpu.async_copy(rows[b], w_sh.at[dst_a.at[ci]], sr[b], add=True)
            pltpu.async_copy(eeb[b], d_sh.at[dst_a.at[ci]], se[b], add=True)

        def wait_scatter(b):
            pltpu.make_async_copy(rows[b], w_sh.at[dst_a.at[0]],
                                  sr[b]).wait()
            pltpu.make_async_copy(eeb[b], d_sh.at[dst_a.at[0]],
                                  se[b]).wait()

        def compute(ci, b, gmv):
            @pl.loop(0, CH // 16)
            def _grp(g):
                sv = src_a[ci, pl.ds(g * 16, 16)]
                dv = dst_a[ci, pl.ds(g * 16, 16)]
                e = plsc.load_gather(el_v, [sv]) + plsc.load_gather(er_v, [dv])
                e = jnp.where(e >= 0.0, e, 0.2 * e)
                ee = jnp.exp(e - gmv)
                eeb[b][pl.ds(g * 16, 16)] = ee
                for l in range(16):
                    wsplat = jnp.broadcast_to(ee[l], (16,))
                    j = g * 16 + l
                    for q in range(H // 16):
                        rows[b][j, pl.ds(q * 16, 16)] = (
                            rows[b][j, pl.ds(q * 16, 16)] * wsplat)

        pltpu.sync_copy(el_hbm.at[0], el_v)
        pltpu.sync_copy(er_hbm.at[0], er_v)
        pltpu.sync_copy(mel_hbm.at[0, pl.ds(0, 16)], gm_v)
        pltpu.sync_copy(mer_hbm.at[0, pl.ds(0, 16)], gm2_v)
        pltpu.sync_copy(ei_hbm.at[pl.ds(wid * NCH, NCH)], src_a)
        pltpu.sync_copy(ei_hbm.at[pl.ds((NC * NS + wid) * NCH, NCH)], dst_a)
        gmv = gm_v[...] + gm2_v[...]

        @pl.loop(0, CH)
        def _z(r):
            for q in range(H // 16):
                rows[0][r, pl.ds(q * 16, 16)] = jnp.zeros((16,), f32)

        for t in range(RPT // CH):
            pltpu.sync_copy(rows[0], w_sh.at[pl.ds(s * RPT + t * CH, CH)])
        for t in range(RPT // H):
            pltpu.sync_copy(rows[0].at[0], d_sh.at[pl.ds(s * RPT + t * H, H)])
        plsc.subcore_barrier()

        issue_gather(0, 0)

        def body(ci, i, r, last):
            b, b1 = r % 3, (r + 1) % 3
            if not last:
                if i is None:
                    wait_scatter(b1)
                else:
                    @pl.when(3 * i + r >= 2)
                    def _():
                        wait_scatter(b1)
                issue_gather(ci + 1, b1)
            wait_gather(b)
            compute(ci, b, gmv)
            issue_scatter(ci, b)

        @pl.loop(0, (NCH - 2) // 3)
        def _i(i):
            for r in range(3):
                body(3 * i + r, i, r, False)

        body(NCH - 2, None, (NCH - 2) % 3, False)
        body(NCH - 1, None, (NCH - 1) % 3, True)
        for b in range(3):
            wait_scatter(b)

        plsc.subcore_barrier()
        pltpu.sync_copy(w_sh.at[pl.ds(s * RPT, RPT)],
                        wout_hbm.at[c, pl.ds(s * RPT, RPT)])
        pltpu.sync_copy(d_sh.at[pl.ds(s * RPT, RPT)],
                        dout_hbm.at[c, pl.ds(s * RPT, RPT)])

    return k(h, el8, er8, mel, mer, edge_index.reshape(2 * E // CH, CH))


def kernel(x, edge_index, W1, al1, ar1, b1, W2, al2, ar2, b2,
           fc1_W, fc1_b, fc2_W, fc2_b):
    x = jnp.pad(x.astype(jnp.float32), ((0, NP - N), (0, 0)))

    h1, el1, er1, mel1, mer1 = _project(x, W1, al1, ar1)
    w1p, d1p = _edge_pass(h1, el1, er1, mel1, mer1, edge_index)
    h2 = _merge(w1p, d1p, b1)

    h2p, el2, er2, mel2, mer2 = _project(h2, W2, al2, ar2)
    w2p, d2p = _edge_pass(h2p, el2, er2, mel2, mer2, edge_index)
    return _head(w2p, d2p, b2, fc1_W, fc1_b, fc2_W, fc2_b)


# speedup vs baseline: 1.4766x; 1.4766x over previous
import dataclasses
import functools

import jax
import jax.numpy as jnp
from jax import lax
from jax.experimental import pallas as pl
from jax.experimental.pallas import tpu as pltpu
from jax.experimental.pallas import tpu_sc as plsc

N = 10000
NP = 10240
E = 320000
H = 64
NC = 2
NS = 16
EPT = E // (NC * NS)
CH = 80
NCH = EPT // CH
RPT = NP // NS

BLK = 1024
GRID = NP // BLK

_SELU_L = 1.0507009873554805
_SELU_A = 1.6732632423543772


def _selu(x):
    return _SELU_L * jnp.where(x > 0, x, _SELU_A * (jnp.exp(x) - 1.0))


def _proj_body(x_ref, w_ref, al_ref, ar_ref, h_ref, el_ref, er_ref, mel_ref, mer_ref):
    i = pl.program_id(0)
    h = jnp.dot(x_ref[...], w_ref[...], preferred_element_type=jnp.float32)
    h_ref[...] = h
    el = jnp.sum(h * al_ref[...][None, :], axis=1)
    er = jnp.sum(h * ar_ref[...][None, :], axis=1)
    el_ref[...] = jnp.broadcast_to(el[None, :], (8, BLK))
    er_ref[...] = jnp.broadcast_to(er[None, :], (8, BLK))
    ml = jnp.full((8, 128), jnp.max(el), dtype=jnp.float32)
    mr = jnp.full((8, 128), jnp.max(er), dtype=jnp.float32)

    @pl.when(i == 0)
    def _():
        mel_ref[...] = ml
        mer_ref[...] = mr

    @pl.when(i > 0)
    def _():
        mel_ref[...] = jnp.maximum(mel_ref[...], ml)
        mer_ref[...] = jnp.maximum(mer_ref[...], mr)


def _project(x, w, al, ar):
    d = x.shape[1]
    f32 = jnp.float32
    return pl.pallas_call(
        _proj_body,
        grid=(GRID,),
        in_specs=[
            pl.BlockSpec((BLK, d), lambda i: (i, 0)),
            pl.BlockSpec((d, H), lambda i: (0, 0)),
            pl.BlockSpec((H,), lambda i: (0,)),
            pl.BlockSpec((H,), lambda i: (0,)),
        ],
        out_specs=[
            pl.BlockSpec((BLK, H), lambda i: (i, 0)),
            pl.BlockSpec((8, BLK), lambda i: (0, i)),
            pl.BlockSpec((8, BLK), lambda i: (0, i)),
            pl.BlockSpec((8, 128), lambda i: (0, 0)),
            pl.BlockSpec((8, 128), lambda i: (0, 0)),
        ],
        out_shape=[
            jax.ShapeDtypeStruct((NP, H), f32),
            jax.ShapeDtypeStruct((8, NP), f32),
            jax.ShapeDtypeStruct((8, NP), f32),
            jax.ShapeDtypeStruct((8, 128), f32),
            jax.ShapeDtypeStruct((8, 128), f32),
        ],
    )(x, w, al, ar)


def _merge_body(wp_ref, dp_ref, b_ref, h_ref):
    w = wp_ref[0] + wp_ref[1]
    dn = dp_ref[0] + dp_ref[1]
    h_ref[...] = _selu(w / (dn[:, None] + 1e-9) + b_ref[...][None, :])


def _merge(wparts, dparts, b):
    return pl.pallas_call(
        _merge_body,
        grid=(GRID,),
        in_specs=[
            pl.BlockSpec((2, BLK, H), lambda i: (0, i, 0)),
            pl.BlockSpec((2, BLK), lambda i: (0, i)),
            pl.BlockSpec((H,), lambda i: (0,)),
        ],
        out_specs=pl.BlockSpec((BLK, H), lambda i: (i, 0)),
        out_shape=jax.ShapeDtypeStruct((NP, H), jnp.float32),
    )(wparts, dparts, b)


def _head_body(wp_ref, dp_ref, b_ref, f1w_ref, f1b_ref, f2w_ref, f2b_ref,
               o_ref, gsum_ref):
    i = pl.program_id(0)
    w = wp_ref[0] + wp_ref[1]
    dn = dp_ref[0] + dp_ref[1]
    h3 = _selu(w / (dn[:, None] + 1e-9) + b_ref[...][None, :])
    row = lax.broadcasted_iota(jnp.int32, (BLK, H), 0) + i * BLK
    h3 = jnp.where(row < N, h3, 0.0)
    s = jnp.broadcast_to(jnp.sum(h3, axis=0)[None, :], (8, H))

    @pl.when(i == 0)
    def _():
        gsum_ref[...] = s

    @pl.when(i > 0)
    def _():
        gsum_ref[...] = gsum_ref[...] + s

    @pl.when(i == GRID - 1)
    def _():
        g = gsum_ref[0:1, :] * (1.0 / N)
        z = _selu(jnp.dot(g, f1w_ref[...], preferred_element_type=jnp.float32)
                  + f1b_ref[...][None, :])
        z = (jnp.dot(z, f2w_ref[...], preferred_element_type=jnp.float32)
             + f2b_ref[...][None, :])
        m = jnp.max(z, axis=1, keepdims=True)
        lse = m + jnp.log(jnp.sum(jnp.exp(z - m), axis=1, keepdims=True))
        o_ref[...] = z - lse


def _head(wparts, dparts, b, f1w, f1b, f2w, f2b):
    h2 = f1w.shape[1]
    c = f2w.shape[1]
    return pl.pallas_call(
        _head_body,
        grid=(GRID,),
        in_specs=[
            pl.BlockSpec((2, BLK, H), lambda i: (0, i, 0)),
            pl.BlockSpec((2, BLK), lambda i: (0, i)),
            pl.BlockSpec((H,), lambda i: (0,)),
            pl.BlockSpec((H, h2), lambda i: (0, 0)),
            pl.BlockSpec((h2,), lambda i: (0,)),
            pl.BlockSpec((h2, c), lambda i: (0, 0)),
            pl.BlockSpec((c,), lambda i: (0,)),
        ],
        out_specs=pl.BlockSpec((1, c), lambda i: (0, 0)),
        out_shape=jax.ShapeDtypeStruct((1, c), jnp.float32),
        scratch_shapes=[pltpu.VMEM((8, H), jnp.float32)],
    )(wparts, dparts, b, f1w, f1b, f2w, f2b)


def _edge_pass(h, el8, er8, mel, mer, edge_index):
    f32 = jnp.float32
    mesh = plsc.VectorSubcoreMesh(core_axis_name="c", subcore_axis_name="s")
    cp = pltpu.CompilerParams()
    if "needs_layout_passes" in pltpu.CompilerParams.__dataclass_fields__:
        cp = dataclasses.replace(cp, needs_layout_passes=False)
    if "use_tc_tiling_on_sc" in pltpu.CompilerParams.__dataclass_fields__:
        cp = dataclasses.replace(cp, use_tc_tiling_on_sc=False)

    @functools.partial(
        pl.kernel,
        compiler_params=cp,
        out_type=[
            jax.ShapeDtypeStruct((NC, NP, H), f32),
            jax.ShapeDtypeStruct((NC, NP), f32),
        ],
        mesh=mesh,
        scratch_types=(
            [
                pltpu.VMEM((NP,), f32),
                pltpu.VMEM((NP,), f32),
                pltpu.VMEM((16,), f32),
                pltpu.VMEM((16,), f32),
                pltpu.VMEM((NCH, CH), jnp.int32),
                pltpu.VMEM((NCH, CH), jnp.int32),
            ]
            + [pltpu.VMEM((CH, H), f32)] * 3
            + [pltpu.VMEM((CH,), f32)] * 3
            + [pltpu.SemaphoreType.DMA] * 9
            + [
                pltpu.VMEM_SHARED((NP, H), f32),
                pltpu.VMEM_SHARED((NP,), f32),
            ]
        ),
    )
    def k(h_hbm, el_hbm, er_hbm, mel_hbm, mer_hbm, ei_hbm, wout_hbm, dout_hbm,
          *scr):
        el_v, er_v, gm_v, gm2_v, src_a, dst_a = scr[0:6]
        rows, eeb = scr[6:9], scr[9:12]
        sg, sr, se = scr[12:15], scr[15:18], scr[18:21]
        w_sh, d_sh = scr[21], scr[22]

        c = lax.axis_index("c")
        s = lax.axis_index("s")
        wid = c * NS + s

        def issue_gather(ci, b):
            pltpu.async_copy(h_hbm.at[src_a.at[ci]], rows[b], sg[b])

        def wait_gather(b):
            pltpu.make_async_copy(h_hbm.at[src_a.at[0]], rows[b],
                                  sg[b]).wait()

        def issue_scatter(ci, b):
            pltpu.async_copy(rows[b], w_sh.at[dst_a.at[ci]], sr[b], add=True)
            pltpu.async_copy(eeb[b], d_sh.at[dst_a.at[ci]], se[b], add=True)

        def wait_scatter(b):
            pltpu.make_async_copy(rows[b], w_sh.at[dst_a.at[0]],
                                  sr[b]).wait()
            pltpu.make_async_copy(eeb[b], d_sh.at[dst_a.at[0]],
                                  se[b]).wait()

        def compute(ci, b, gmv):
            @pl.loop(0, CH // 16)
            def _grp(g):
                sv = src_a[ci, pl.ds(g * 16, 16)]
                dv = dst_a[ci, pl.ds(g * 16, 16)]
                e = plsc.load_gather(el_v, [sv]) + plsc.load_gather(er_v, [dv])
                e = jnp.where(e >= 0.0, e, 0.2 * e)
                eeb[b][pl.ds(g * 16, 16)] = jnp.exp(e - gmv)

            @pl.loop(0, CH, unroll=8)
            def _edge(j):
                wsplat = plsc.load_gather(
                    eeb[b], [jnp.full((16,), 0, jnp.int32) + j])
                for q in range(H // 16):
                    rows[b][j, pl.ds(q * 16, 16)] = (
                        rows[b][j, pl.ds(q * 16, 16)] * wsplat)

        pltpu.sync_copy(el_hbm.at[0], el_v)
        pltpu.sync_copy(er_hbm.at[0], er_v)
        pltpu.sync_copy(mel_hbm.at[0, pl.ds(0, 16)], gm_v)
        pltpu.sync_copy(mer_hbm.at[0, pl.ds(0, 16)], gm2_v)
        pltpu.sync_copy(ei_hbm.at[pl.ds(wid * NCH, NCH)], src_a)
        pltpu.sync_copy(ei_hbm.at[pl.ds((NC * NS + wid) * NCH, NCH)], dst_a)
        gmv = gm_v[...] + gm2_v[...]

        @pl.loop(0, CH)
        def _z(r):
            for q in range(H // 16):
                rows[0][r, pl.ds(q * 16, 16)] = jnp.zeros((16,), f32)

        for t in range(RPT // CH):
            pltpu.sync_copy(rows[0], w_sh.at[pl.ds(s * RPT + t * CH, CH)])
        for t in range(RPT // H):
            pltpu.sync_copy(rows[0].at[0], d_sh.at[pl.ds(s * RPT + t * H, H)])
        plsc.subcore_barrier()

        issue_gather(0, 0)

        def body(ci, i, r, last):
            b, b1 = r % 3, (r + 1) % 3
            if not last:
                if i is None:
                    wait_scatter(b1)
                else:
                    @pl.when(3 * i + r >= 2)
                    def _():
                        wait_scatter(b1)
                issue_gather(ci + 1, b1)
            wait_gather(b)
            compute(ci, b, gmv)
            issue_scatter(ci, b)

        @pl.loop(0, (NCH - 2) // 3)
        def _i(i):
            for r in range(3):
                body(3 * i + r, i, r, False)

        body(NCH - 2, None, (NCH - 2) % 3, False)
        body(NCH - 1, None, (NCH - 1) % 3, True)
        for b in range(3):
            wait_scatter(b)

        plsc.subcore_barrier()
        pltpu.sync_copy(w_sh.at[pl.ds(s * RPT, RPT)],
                        wout_hbm.at[c, pl.ds(s * RPT, RPT)])
        pltpu.sync_copy(d_sh.at[pl.ds(s * RPT, RPT)],
                        dout_hbm.at[c, pl.ds(s * RPT, RPT)])

    return k(h, el8, er8, mel, mer, edge_index.reshape(2 * E // CH, CH))


def kernel(x, edge_index, W1, al1, ar1, b1, W2, al2, ar2, b2,
           fc1_W, fc1_b, fc2_W, fc2_b):
    x = jnp.pad(x.astype(jnp.float32), ((0, NP - N), (0, 0)))

    h1, el1, er1, mel1, mer1 = _project(x, W1, al1, ar1)
    w1p, d1p = _edge_pass(h1, el1, er1, mel1, mer1, edge_index)
    h2 = _merge(w1p, d1p, b1)

    h2p, el2, er2, mel2, mer2 = _project(h2, W2, al2, ar2)
    w2p, d2p = _edge_pass(h2p, el2, er2, mel2, mer2, edge_index)
    return _head(w2p, d2p, b2, fc1_W, fc1_b, fc2_W, fc2_b)


# speedup vs baseline: 1.5140x; 1.0253x over previous
import dataclasses
import functools

import jax
import jax.numpy as jnp
from jax import lax
from jax.experimental import pallas as pl
from jax.experimental.pallas import tpu as pltpu
from jax.experimental.pallas import tpu_sc as plsc

N = 10000
NP = 10240
E = 320000
H = 64
NC = 2
NS = 16
EPT = E // (NC * NS)
CH = 80
NCH = EPT // CH
RPT = NP // NS

BLK = 1024
GRID = NP // BLK

_SELU_L = 1.0507009873554805
_SELU_A = 1.6732632423543772


def _selu(x):
    return _SELU_L * jnp.where(x > 0, x, _SELU_A * (jnp.exp(x) - 1.0))


def _proj_body(x_ref, w_ref, al_ref, ar_ref, h_ref, el_ref, er_ref, mel_ref, mer_ref):
    i = pl.program_id(0)
    h = jnp.dot(x_ref[...], w_ref[...], preferred_element_type=jnp.float32)
    h_ref[...] = h
    el = jnp.sum(h * al_ref[...][None, :], axis=1)
    er = jnp.sum(h * ar_ref[...][None, :], axis=1)
    el_ref[...] = jnp.broadcast_to(el[None, :], (8, BLK))
    er_ref[...] = jnp.broadcast_to(er[None, :], (8, BLK))
    ml = jnp.full((8, 128), jnp.max(el), dtype=jnp.float32)
    mr = jnp.full((8, 128), jnp.max(er), dtype=jnp.float32)

    @pl.when(i == 0)
    def _():
        mel_ref[...] = ml
        mer_ref[...] = mr

    @pl.when(i > 0)
    def _():
        mel_ref[...] = jnp.maximum(mel_ref[...], ml)
        mer_ref[...] = jnp.maximum(mer_ref[...], mr)


def _project(x, w, al, ar):
    d = x.shape[1]
    f32 = jnp.float32
    return pl.pallas_call(
        _proj_body,
        grid=(GRID,),
        in_specs=[
            pl.BlockSpec((BLK, d), lambda i: (i, 0)),
            pl.BlockSpec((d, H), lambda i: (0, 0)),
            pl.BlockSpec((H,), lambda i: (0,)),
            pl.BlockSpec((H,), lambda i: (0,)),
        ],
        out_specs=[
            pl.BlockSpec((BLK, H), lambda i: (i, 0)),
            pl.BlockSpec((8, BLK), lambda i: (0, i)),
            pl.BlockSpec((8, BLK), lambda i: (0, i)),
            pl.BlockSpec((8, 128), lambda i: (0, 0)),
            pl.BlockSpec((8, 128), lambda i: (0, 0)),
        ],
        out_shape=[
            jax.ShapeDtypeStruct((NP, H), f32),
            jax.ShapeDtypeStruct((8, NP), f32),
            jax.ShapeDtypeStruct((8, NP), f32),
            jax.ShapeDtypeStruct((8, 128), f32),
            jax.ShapeDtypeStruct((8, 128), f32),
        ],
    )(x, w, al, ar)


def _merge_body(wp_ref, dp_ref, b_ref, h_ref):
    w = wp_ref[0] + wp_ref[1]
    dn = dp_ref[0] + dp_ref[1]
    h_ref[...] = _selu(w / (dn[:, None] + 1e-9) + b_ref[...][None, :])


def _merge(wparts, dparts, b):
    return pl.pallas_call(
        _merge_body,
        grid=(GRID,),
        in_specs=[
            pl.BlockSpec((2, BLK, H), lambda i: (0, i, 0)),
            pl.BlockSpec((2, BLK), lambda i: (0, i)),
            pl.BlockSpec((H,), lambda i: (0,)),
        ],
        out_specs=pl.BlockSpec((BLK, H), lambda i: (i, 0)),
        out_shape=jax.ShapeDtypeStruct((NP, H), jnp.float32),
    )(wparts, dparts, b)


def _merge_proj_body(wp_ref, dp_ref, b_ref, w_ref, al_ref, ar_ref,
                     h_ref, el_ref, er_ref, mel_ref, mer_ref):
    i = pl.program_id(0)
    wsum = wp_ref[0] + wp_ref[1]
    dn = dp_ref[0] + dp_ref[1]
    h2 = _selu(wsum / (dn[:, None] + 1e-9) + b_ref[...][None, :])
    h = jnp.dot(h2, w_ref[...], preferred_element_type=jnp.float32)
    h_ref[...] = h
    el = jnp.sum(h * al_ref[...][None, :], axis=1)
    er = jnp.sum(h * ar_ref[...][None, :], axis=1)
    el_ref[...] = jnp.broadcast_to(el[None, :], (8, BLK))
    er_ref[...] = jnp.broadcast_to(er[None, :], (8, BLK))
    ml = jnp.full((8, 128), jnp.max(el), dtype=jnp.float32)
    mr = jnp.full((8, 128), jnp.max(er), dtype=jnp.float32)

    @pl.when(i == 0)
    def _():
        mel_ref[...] = ml
        mer_ref[...] = mr

    @pl.when(i > 0)
    def _():
        mel_ref[...] = jnp.maximum(mel_ref[...], ml)
        mer_ref[...] = jnp.maximum(mer_ref[...], mr)


def _merge_project(wparts, dparts, b, w, al, ar):
    f32 = jnp.float32
    return pl.pallas_call(
        _merge_proj_body,
        grid=(GRID,),
        in_specs=[
            pl.BlockSpec((2, BLK, H), lambda i: (0, i, 0)),
            pl.BlockSpec((2, BLK), lambda i: (0, i)),
            pl.BlockSpec((H,), lambda i: (0,)),
            pl.BlockSpec((H, H), lambda i: (0, 0)),
            pl.BlockSpec((H,), lambda i: (0,)),
            pl.BlockSpec((H,), lambda i: (0,)),
        ],
        out_specs=[
            pl.BlockSpec((BLK, H), lambda i: (i, 0)),
            pl.BlockSpec((8, BLK), lambda i: (0, i)),
            pl.BlockSpec((8, BLK), lambda i: (0, i)),
            pl.BlockSpec((8, 128), lambda i: (0, 0)),
            pl.BlockSpec((8, 128), lambda i: (0, 0)),
        ],
        out_shape=[
            jax.ShapeDtypeStruct((NP, H), f32),
            jax.ShapeDtypeStruct((8, NP), f32),
            jax.ShapeDtypeStruct((8, NP), f32),
            jax.ShapeDtypeStruct((8, 128), f32),
            jax.ShapeDtypeStruct((8, 128), f32),
        ],
    )(wparts, dparts, b, w, al, ar)


def _head_body(wp_ref, dp_ref, b_ref, f1w_ref, f1b_ref, f2w_ref, f2b_ref,
               o_ref, gsum_ref):
    i = pl.program_id(0)
    w = wp_ref[0] + wp_ref[1]
    dn = dp_ref[0] + dp_ref[1]
    h3 = _selu(w / (dn[:, None] + 1e-9) + b_ref[...][None, :])
    row = lax.broadcasted_iota(jnp.int32, (BLK, H), 0) + i * BLK
    h3 = jnp.where(row < N, h3, 0.0)
    s = jnp.broadcast_to(jnp.sum(h3, axis=0)[None, :], (8, H))

    @pl.when(i == 0)
    def _():
        gsum_ref[...] = s

    @pl.when(i > 0)
    def _():
        gsum_ref[...] = gsum_ref[...] + s

    @pl.when(i == GRID - 1)
    def _():
        g = gsum_ref[0:1, :] * (1.0 / N)
        z = _selu(jnp.dot(g, f1w_ref[...], preferred_element_type=jnp.float32)
                  + f1b_ref[...][None, :])
        z = (jnp.dot(z, f2w_ref[...], preferred_element_type=jnp.float32)
             + f2b_ref[...][None, :])
        m = jnp.max(z, axis=1, keepdims=True)
        lse = m + jnp.log(jnp.sum(jnp.exp(z - m), axis=1, keepdims=True))
        o_ref[...] = z - lse


def _head(wparts, dparts, b, f1w, f1b, f2w, f2b):
    h2 = f1w.shape[1]
    c = f2w.shape[1]
    return pl.pallas_call(
        _head_body,
        grid=(GRID,),
        in_specs=[
            pl.BlockSpec((2, BLK, H), lambda i: (0, i, 0)),
            pl.BlockSpec((2, BLK), lambda i: (0, i)),
            pl.BlockSpec((H,), lambda i: (0,)),
            pl.BlockSpec((H, h2), lambda i: (0, 0)),
            pl.BlockSpec((h2,), lambda i: (0,)),
            pl.BlockSpec((h2, c), lambda i: (0, 0)),
            pl.BlockSpec((c,), lambda i: (0,)),
        ],
        out_specs=pl.BlockSpec((1, c), lambda i: (0, 0)),
        out_shape=jax.ShapeDtypeStruct((1, c), jnp.float32),
        scratch_shapes=[pltpu.VMEM((8, H), jnp.float32)],
    )(wparts, dparts, b, f1w, f1b, f2w, f2b)


def _edge_pass(h, el8, er8, mel, mer, edge_index):
    f32 = jnp.float32
    mesh = plsc.VectorSubcoreMesh(core_axis_name="c", subcore_axis_name="s")
    cp = pltpu.CompilerParams()
    if "needs_layout_passes" in pltpu.CompilerParams.__dataclass_fields__:
        cp = dataclasses.replace(cp, needs_layout_passes=False)
    if "use_tc_tiling_on_sc" in pltpu.CompilerParams.__dataclass_fields__:
        cp = dataclasses.replace(cp, use_tc_tiling_on_sc=False)

    @functools.partial(
        pl.kernel,
        compiler_params=cp,
        out_type=[
            jax.ShapeDtypeStruct((NC, NP, H), f32),
            jax.ShapeDtypeStruct((NC, NP), f32),
        ],
        mesh=mesh,
        scratch_types=(
            [
                pltpu.VMEM((NP,), f32),
                pltpu.VMEM((NP,), f32),
                pltpu.VMEM((16,), f32),
                pltpu.VMEM((16,), f32),
                pltpu.VMEM((NCH, CH), jnp.int32),
                pltpu.VMEM((NCH, CH), jnp.int32),
            ]
            + [pltpu.VMEM((CH, H), f32)] * 3
            + [pltpu.VMEM((CH,), f32)] * 3
            + [pltpu.SemaphoreType.DMA] * 9
            + [
                pltpu.VMEM_SHARED((NP, H), f32),
                pltpu.VMEM_SHARED((NP,), f32),
            ]
        ),
    )
    def k(h_hbm, el_hbm, er_hbm, mel_hbm, mer_hbm, ei_hbm, wout_hbm, dout_hbm,
          *scr):
        el_v, er_v, gm_v, gm2_v, src_a, dst_a = scr[0:6]
        rows, eeb = scr[6:9], scr[9:12]
        sg, sr, se = scr[12:15], scr[15:18], scr[18:21]
        w_sh, d_sh = scr[21], scr[22]

        c = lax.axis_index("c")
        s = lax.axis_index("s")
        wid = c * NS + s

        def issue_gather(ci, b):
            pltpu.async_copy(h_hbm.at[src_a.at[ci]], rows[b], sg[b])

        def wait_gather(b):
            pltpu.make_async_copy(h_hbm.at[src_a.at[0]], rows[b],
                                  sg[b]).wait()

        def issue_scatter(ci, b):
            pltpu.async_copy(rows[b], w_sh.at[dst_a.at[ci]], sr[b], add=True)
            pltpu.async_copy(eeb[b], d_sh.at[dst_a.at[ci]], se[b], add=True)

        def wait_scatter(b):
            pltpu.make_async_copy(rows[b], w_sh.at[dst_a.at[0]],
                                  sr[b]).wait()
            pltpu.make_async_copy(eeb[b], d_sh.at[dst_a.at[0]],
                                  se[b]).wait()

        def compute(ci, b, gmv):
            @pl.loop(0, CH // 16)
            def _grp(g):
                sv = src_a[ci, pl.ds(g * 16, 16)]
                dv = dst_a[ci, pl.ds(g * 16, 16)]
                e = plsc.load_gather(el_v, [sv]) + plsc.load_gather(er_v, [dv])
                e = jnp.where(e >= 0.0, e, 0.2 * e)
                eeb[b][pl.ds(g * 16, 16)] = jnp.exp(e - gmv)

            @pl.loop(0, CH, unroll=8)
            def _edge(j):
                wsplat = plsc.load_gather(
                    eeb[b], [jnp.full((16,), 0, jnp.int32) + j])
                for q in range(H // 16):
                    rows[b][j, pl.ds(q * 16, 16)] = (
                        rows[b][j, pl.ds(q * 16, 16)] * wsplat)

        pltpu.sync_copy(el_hbm.at[0], el_v)
        pltpu.sync_copy(er_hbm.at[0], er_v)
        pltpu.sync_copy(mel_hbm.at[0, pl.ds(0, 16)], gm_v)
        pltpu.sync_copy(mer_hbm.at[0, pl.ds(0, 16)], gm2_v)
        pltpu.sync_copy(ei_hbm.at[pl.ds(wid * NCH, NCH)], src_a)
        pltpu.sync_copy(ei_hbm.at[pl.ds((NC * NS + wid) * NCH, NCH)], dst_a)
        gmv = gm_v[...] + gm2_v[...]

        @pl.loop(0, CH)
        def _z(r):
            for q in range(H // 16):
                rows[0][r, pl.ds(q * 16, 16)] = jnp.zeros((16,), f32)

        for t in range(RPT // CH):
            pltpu.sync_copy(rows[0], w_sh.at[pl.ds(s * RPT + t * CH, CH)])
        for t in range(RPT // H):
            pltpu.sync_copy(rows[0].at[0], d_sh.at[pl.ds(s * RPT + t * H, H)])
        plsc.subcore_barrier()

        issue_gather(0, 0)

        def body(ci, i, r, last):
            b, b1 = r % 3, (r + 1) % 3
            if not last:
                if i is None:
                    wait_scatter(b1)
                else:
                    @pl.when(3 * i + r >= 2)
                    def _():
                        wait_scatter(b1)
                issue_gather(ci + 1, b1)
            wait_gather(b)
            compute(ci, b, gmv)
            issue_scatter(ci, b)

        @pl.loop(0, (NCH - 2) // 3)
        def _i(i):
            for r in range(3):
                body(3 * i + r, i, r, False)

        body(NCH - 2, None, (NCH - 2) % 3, False)
        body(NCH - 1, None, (NCH - 1) % 3, True)
        for b in range(3):
            wait_scatter(b)

        plsc.subcore_barrier()
        pltpu.sync_copy(w_sh.at[pl.ds(s * RPT, RPT)],
                        wout_hbm.at[c, pl.ds(s * RPT, RPT)])
        pltpu.sync_copy(d_sh.at[pl.ds(s * RPT, RPT)],
                        dout_hbm.at[c, pl.ds(s * RPT, RPT)])

    return k(h, el8, er8, mel, mer, edge_index.reshape(2 * E // CH, CH))


def kernel(x, edge_index, W1, al1, ar1, b1, W2, al2, ar2, b2,
           fc1_W, fc1_b, fc2_W, fc2_b):
    x = jnp.pad(x.astype(jnp.float32), ((0, NP - N), (0, 0)))

    h1, el1, er1, mel1, mer1 = _project(x, W1, al1, ar1)
    w1p, d1p = _edge_pass(h1, el1, er1, mel1, mer1, edge_index)
    h2p, el2, er2, mel2, mer2 = _merge_project(w1p, d1p, b1, W2, al2, ar2)
    w2p, d2p = _edge_pass(h2p, el2, er2, mel2, mer2, edge_index)
    return _head(w2p, d2p, b2, fc1_W, fc1_b, fc2_W, fc2_b)


# speedup vs baseline: 1.7486x; 1.1550x over previous
import dataclasses
import functools

import jax
import jax.numpy as jnp
from jax import lax
from jax.experimental import pallas as pl
from jax.experimental.pallas import tpu as pltpu
from jax.experimental.pallas import tpu_sc as plsc

N = 10000
NP = 10240
E = 320000
H = 64
NC = 2
NS = 16
EPT = E // (NC * NS)
CH = 80
NCH = EPT // CH
RPT = NP // NS

BLK = 1024
GRID = NP // BLK

_SELU_L = 1.0507009873554805
_SELU_A = 1.6732632423543772


def _selu(x):
    return _SELU_L * jnp.where(x > 0, x, _SELU_A * (jnp.exp(x) - 1.0))


def _proj_body(x_ref, w_ref, al_ref, ar_ref, h_ref, el_ref, er_ref, mel_ref, mer_ref):
    i = pl.program_id(0)
    h = jnp.dot(x_ref[...], w_ref[...], preferred_element_type=jnp.float32)
    h_ref[...] = h
    el = jnp.sum(h * al_ref[...][None, :], axis=1)
    er = jnp.sum(h * ar_ref[...][None, :], axis=1)
    el_ref[...] = jnp.broadcast_to(el[None, :], (8, BLK))
    er_ref[...] = jnp.broadcast_to(er[None, :], (8, BLK))
    ml = jnp.full((8, 128), jnp.max(el), dtype=jnp.float32)
    mr = jnp.full((8, 128), jnp.max(er), dtype=jnp.float32)

    @pl.when(i == 0)
    def _():
        mel_ref[...] = ml
        mer_ref[...] = mr

    @pl.when(i > 0)
    def _():
        mel_ref[...] = jnp.maximum(mel_ref[...], ml)
        mer_ref[...] = jnp.maximum(mer_ref[...], mr)


def _project(x, w, al, ar):
    d = x.shape[1]
    f32 = jnp.float32
    return pl.pallas_call(
        _proj_body,
        grid=(GRID,),
        in_specs=[
            pl.BlockSpec((BLK, d), lambda i: (i, 0)),
            pl.BlockSpec((d, H), lambda i: (0, 0)),
            pl.BlockSpec((H,), lambda i: (0,)),
            pl.BlockSpec((H,), lambda i: (0,)),
        ],
        out_specs=[
            pl.BlockSpec((BLK, H), lambda i: (i, 0)),
            pl.BlockSpec((8, BLK), lambda i: (0, i)),
            pl.BlockSpec((8, BLK), lambda i: (0, i)),
            pl.BlockSpec((8, 128), lambda i: (0, 0)),
            pl.BlockSpec((8, 128), lambda i: (0, 0)),
        ],
        out_shape=[
            jax.ShapeDtypeStruct((NP, H), f32),
            jax.ShapeDtypeStruct((8, NP), f32),
            jax.ShapeDtypeStruct((8, NP), f32),
            jax.ShapeDtypeStruct((8, 128), f32),
            jax.ShapeDtypeStruct((8, 128), f32),
        ],
    )(x, w, al, ar)


def _merge_body(wp_ref, dp_ref, b_ref, h_ref):
    w = wp_ref[0] + wp_ref[1]
    dn = dp_ref[0] + dp_ref[1]
    h_ref[...] = _selu(w / (dn[:, None] + 1e-9) + b_ref[...][None, :])


def _merge(wparts, dparts, b):
    return pl.pallas_call(
        _merge_body,
        grid=(GRID,),
        in_specs=[
            pl.BlockSpec((2, BLK, H), lambda i: (0, i, 0)),
            pl.BlockSpec((2, BLK), lambda i: (0, i)),
            pl.BlockSpec((H,), lambda i: (0,)),
        ],
        out_specs=pl.BlockSpec((BLK, H), lambda i: (i, 0)),
        out_shape=jax.ShapeDtypeStruct((NP, H), jnp.float32),
    )(wparts, dparts, b)


def _merge_proj_body(wp_ref, dp_ref, b_ref, w_ref, al_ref, ar_ref,
                     h_ref, el_ref, er_ref, mel_ref, mer_ref):
    i = pl.program_id(0)
    wsum = wp_ref[0] + wp_ref[1]
    dn = dp_ref[0] + dp_ref[1]
    h2 = _selu(wsum / (dn[:, None] + 1e-9) + b_ref[...][None, :])
    h = jnp.dot(h2, w_ref[...], preferred_element_type=jnp.float32)
    h_ref[...] = h
    el = jnp.sum(h * al_ref[...][None, :], axis=1)
    er = jnp.sum(h * ar_ref[...][None, :], axis=1)
    el_ref[...] = jnp.broadcast_to(el[None, :], (8, BLK))
    er_ref[...] = jnp.broadcast_to(er[None, :], (8, BLK))
    ml = jnp.full((8, 128), jnp.max(el), dtype=jnp.float32)
    mr = jnp.full((8, 128), jnp.max(er), dtype=jnp.float32)

    @pl.when(i == 0)
    def _():
        mel_ref[...] = ml
        mer_ref[...] = mr

    @pl.when(i > 0)
    def _():
        mel_ref[...] = jnp.maximum(mel_ref[...], ml)
        mer_ref[...] = jnp.maximum(mer_ref[...], mr)


def _merge_project(wparts, dparts, b, w, al, ar):
    f32 = jnp.float32
    return pl.pallas_call(
        _merge_proj_body,
        grid=(GRID,),
        in_specs=[
            pl.BlockSpec((2, BLK, H), lambda i: (0, i, 0)),
            pl.BlockSpec((2, BLK), lambda i: (0, i)),
            pl.BlockSpec((H,), lambda i: (0,)),
            pl.BlockSpec((H, H), lambda i: (0, 0)),
            pl.BlockSpec((H,), lambda i: (0,)),
            pl.BlockSpec((H,), lambda i: (0,)),
        ],
        out_specs=[
            pl.BlockSpec((BLK, H), lambda i: (i, 0)),
            pl.BlockSpec((8, BLK), lambda i: (0, i)),
            pl.BlockSpec((8, BLK), lambda i: (0, i)),
            pl.BlockSpec((8, 128), lambda i: (0, 0)),
            pl.BlockSpec((8, 128), lambda i: (0, 0)),
        ],
        out_shape=[
            jax.ShapeDtypeStruct((NP, H), f32),
            jax.ShapeDtypeStruct((8, NP), f32),
            jax.ShapeDtypeStruct((8, NP), f32),
            jax.ShapeDtypeStruct((8, 128), f32),
            jax.ShapeDtypeStruct((8, 128), f32),
        ],
    )(wparts, dparts, b, w, al, ar)


def _head_body(wp_ref, dp_ref, b_ref, f1w_ref, f1b_ref, f2w_ref, f2b_ref,
               o_ref, gsum_ref):
    i = pl.program_id(0)
    w = wp_ref[0] + wp_ref[1]
    dn = dp_ref[0] + dp_ref[1]
    h3 = _selu(w / (dn[:, None] + 1e-9) + b_ref[...][None, :])
    row = lax.broadcasted_iota(jnp.int32, (BLK, H), 0) + i * BLK
    h3 = jnp.where(row < N, h3, 0.0)
    s = jnp.broadcast_to(jnp.sum(h3, axis=0)[None, :], (8, H))

    @pl.when(i == 0)
    def _():
        gsum_ref[...] = s

    @pl.when(i > 0)
    def _():
        gsum_ref[...] = gsum_ref[...] + s

    @pl.when(i == GRID - 1)
    def _():
        g = gsum_ref[0:1, :] * (1.0 / N)
        z = _selu(jnp.dot(g, f1w_ref[...], preferred_element_type=jnp.float32)
                  + f1b_ref[...][None, :])
        z = (jnp.dot(z, f2w_ref[...], preferred_element_type=jnp.float32)
             + f2b_ref[...][None, :])
        m = jnp.max(z, axis=1, keepdims=True)
        lse = m + jnp.log(jnp.sum(jnp.exp(z - m), axis=1, keepdims=True))
        o_ref[...] = z - lse


def _head(wparts, dparts, b, f1w, f1b, f2w, f2b):
    h2 = f1w.shape[1]
    c = f2w.shape[1]
    return pl.pallas_call(
        _head_body,
        grid=(GRID,),
        in_specs=[
            pl.BlockSpec((2, BLK, H), lambda i: (0, i, 0)),
            pl.BlockSpec((2, BLK), lambda i: (0, i)),
            pl.BlockSpec((H,), lambda i: (0,)),
            pl.BlockSpec((H, h2), lambda i: (0, 0)),
            pl.BlockSpec((h2,), lambda i: (0,)),
            pl.BlockSpec((h2, c), lambda i: (0, 0)),
            pl.BlockSpec((c,), lambda i: (0,)),
        ],
        out_specs=pl.BlockSpec((1, c), lambda i: (0, 0)),
        out_shape=jax.ShapeDtypeStruct((1, c), jnp.float32),
        scratch_shapes=[pltpu.VMEM((8, H), jnp.float32)],
    )(wparts, dparts, b, f1w, f1b, f2w, f2b)


def _edge_pass(h, el8, er8, mel, mer, edge_index):
    f32 = jnp.float32
    mesh = plsc.VectorSubcoreMesh(core_axis_name="c", subcore_axis_name="s")
    cp = pltpu.CompilerParams()
    if "needs_layout_passes" in pltpu.CompilerParams.__dataclass_fields__:
        cp = dataclasses.replace(cp, needs_layout_passes=False)
    if "use_tc_tiling_on_sc" in pltpu.CompilerParams.__dataclass_fields__:
        cp = dataclasses.replace(cp, use_tc_tiling_on_sc=False)

    @functools.partial(
        pl.kernel,
        compiler_params=cp,
        out_type=[
            jax.ShapeDtypeStruct((NC, NP, H), f32),
            jax.ShapeDtypeStruct((NC, NP), f32),
        ],
        mesh=mesh,
        scratch_types=(
            [
                pltpu.VMEM((NP,), f32),
                pltpu.VMEM((NP,), f32),
                pltpu.VMEM((16,), f32),
                pltpu.VMEM((16,), f32),
                pltpu.VMEM((NCH, CH), jnp.int32),
                pltpu.VMEM((NCH, CH), jnp.int32),
            ]
            + [pltpu.VMEM((CH, H), f32)] * 3
            + [pltpu.VMEM((CH,), f32)] * 3
            + [pltpu.SemaphoreType.DMA] * 9
            + [
                pltpu.VMEM_SHARED((NP, H), f32),
                pltpu.VMEM_SHARED((NP,), f32),
            ]
        ),
    )
    def k(h_hbm, el_hbm, er_hbm, mel_hbm, mer_hbm, ei_hbm, wout_hbm, dout_hbm,
          *scr):
        el_v, er_v, gm_v, gm2_v, src_a, dst_a = scr[0:6]
        rows, eeb = scr[6:9], scr[9:12]
        sg, sr, se = scr[12:15], scr[15:18], scr[18:21]
        w_sh, d_sh = scr[21], scr[22]

        c = lax.axis_index("c")
        s = lax.axis_index("s")
        wid = c * NS + s

        def issue_gather(ci, b):
            pltpu.async_copy(h_hbm.at[src_a.at[ci]], rows[b], sg[b])

        def wait_gather(b):
            pltpu.make_async_copy(h_hbm.at[src_a.at[0]], rows[b],
                                  sg[b]).wait()

        def issue_scatter(ci, b):
            pltpu.async_copy(rows[b], w_sh.at[dst_a.at[ci]], sr[b], add=True)
            pltpu.async_copy(eeb[b], d_sh.at[dst_a.at[ci]], se[b], add=True)

        def wait_scatter(b):
            pltpu.make_async_copy(rows[b], w_sh.at[dst_a.at[0]],
                                  sr[b]).wait()
            pltpu.make_async_copy(eeb[b], d_sh.at[dst_a.at[0]],
                                  se[b]).wait()

        def compute(ci, b, gmv):
            @pl.loop(0, CH // 16)
            def _grp(g):
                sv = src_a[ci, pl.ds(g * 16, 16)]
                dv = dst_a[ci, pl.ds(g * 16, 16)]
                e = plsc.load_gather(el_v, [sv]) + plsc.load_gather(er_v, [dv])
                e = jnp.where(e >= 0.0, e, 0.2 * e)
                eeb[b][pl.ds(g * 16, 16)] = jnp.exp(e - gmv)

            @plsc.parallel_loop(0, CH, 1, unroll=8)
            def _edge(j):
                wsplat = plsc.load_gather(
                    eeb[b], [jnp.full((16,), 0, jnp.int32) + j])
                for q in range(H // 16):
                    rows[b][j, pl.ds(q * 16, 16)] = (
                        rows[b][j, pl.ds(q * 16, 16)] * wsplat)

        pltpu.sync_copy(el_hbm.at[0], el_v)
        pltpu.sync_copy(er_hbm.at[0], er_v)
        pltpu.sync_copy(mel_hbm.at[0, pl.ds(0, 16)], gm_v)
        pltpu.sync_copy(mer_hbm.at[0, pl.ds(0, 16)], gm2_v)
        pltpu.sync_copy(ei_hbm.at[pl.ds(wid * NCH, NCH)], src_a)
        pltpu.sync_copy(ei_hbm.at[pl.ds((NC * NS + wid) * NCH, NCH)], dst_a)
        gmv = gm_v[...] + gm2_v[...]

        @pl.loop(0, CH)
        def _z(r):
            for q in range(H // 16):
                rows[0][r, pl.ds(q * 16, 16)] = jnp.zeros((16,), f32)

        for t in range(RPT // CH):
            pltpu.sync_copy(rows[0], w_sh.at[pl.ds(s * RPT + t * CH, CH)])
        for t in range(RPT // H):
            pltpu.sync_copy(rows[0].at[0], d_sh.at[pl.ds(s * RPT + t * H, H)])
        plsc.subcore_barrier()

        issue_gather(0, 0)

        def body(ci, i, r, last):
            b, b1 = r % 3, (r + 1) % 3
            if not last:
                if i is None:
                    wait_scatter(b1)
                else:
                    @pl.when(3 * i + r >= 2)
                    def _():
                        wait_scatter(b1)
                issue_gather(ci + 1, b1)
            wait_gather(b)
            compute(ci, b, gmv)
            issue_scatter(ci, b)

        @pl.loop(0, (NCH - 2) // 3)
        def _i(i):
            for r in range(3):
                body(3 * i + r, i, r, False)

        body(NCH - 2, None, (NCH - 2) % 3, False)
        body(NCH - 1, None, (NCH - 1) % 3, True)
        for b in range(3):
            wait_scatter(b)

        plsc.subcore_barrier()
        pltpu.sync_copy(w_sh.at[pl.ds(s * RPT, RPT)],
                        wout_hbm.at[c, pl.ds(s * RPT, RPT)])
        pltpu.sync_copy(d_sh.at[pl.ds(s * RPT, RPT)],
                        dout_hbm.at[c, pl.ds(s * RPT, RPT)])

    return k(h, el8, er8, mel, mer, edge_index.reshape(2 * E // CH, CH))


def kernel(x, edge_index, W1, al1, ar1, b1, W2, al2, ar2, b2,
           fc1_W, fc1_b, fc2_W, fc2_b):
    x = jnp.pad(x.astype(jnp.float32), ((0, NP - N), (0, 0)))

    h1, el1, er1, mel1, mer1 = _project(x, W1, al1, ar1)
    w1p, d1p = _edge_pass(h1, el1, er1, mel1, mer1, edge_index)
    h2p, el2, er2, mel2, mer2 = _merge_project(w1p, d1p, b1, W2, al2, ar2)
    w2p, d2p = _edge_pass(h2p, el2, er2, mel2, mer2, edge_index)
    return _head(w2p, d2p, b2, fc1_W, fc1_b, fc2_W, fc2_b)


# speedup vs baseline: 1.8103x; 1.0353x over previous
import dataclasses
import functools

import jax
import jax.numpy as jnp
from jax import lax
from jax.experimental import pallas as pl
from jax.experimental.pallas import tpu as pltpu
from jax.experimental.pallas import tpu_sc as plsc

N = 10000
NP = 10240
E = 320000
H = 64
NC = 2
NS = 16
EPT = E // (NC * NS)
CH = 80
NCH = EPT // CH
RPT = NP // NS

BLK = 1024
GRID = NP // BLK

_SELU_L = 1.0507009873554805
_SELU_A = 1.6732632423543772


def _selu(x):
    return _SELU_L * jnp.where(x > 0, x, _SELU_A * (jnp.exp(x) - 1.0))


def _proj_body(x_ref, w_ref, al_ref, ar_ref, h_ref, el_ref, er_ref, mel_ref, mer_ref):
    i = pl.program_id(0)
    h = jnp.dot(x_ref[...], w_ref[...], preferred_element_type=jnp.float32)
    h_ref[...] = h
    el = jnp.sum(h * al_ref[...][None, :], axis=1)
    er = jnp.sum(h * ar_ref[...][None, :], axis=1)
    el_ref[...] = jnp.broadcast_to(el[None, :], (8, BLK))
    er_ref[...] = jnp.broadcast_to(er[None, :], (8, BLK))
    ml = jnp.full((8, 128), jnp.max(el), dtype=jnp.float32)
    mr = jnp.full((8, 128), jnp.max(er), dtype=jnp.float32)

    @pl.when(i == 0)
    def _():
        mel_ref[...] = ml
        mer_ref[...] = mr

    @pl.when(i > 0)
    def _():
        mel_ref[...] = jnp.maximum(mel_ref[...], ml)
        mer_ref[...] = jnp.maximum(mer_ref[...], mr)


def _project(x, w, al, ar):
    d = x.shape[1]
    f32 = jnp.float32
    return pl.pallas_call(
        _proj_body,
        grid=(GRID,),
        in_specs=[
            pl.BlockSpec((BLK, d), lambda i: (i, 0)),
            pl.BlockSpec((d, H), lambda i: (0, 0)),
            pl.BlockSpec((H,), lambda i: (0,)),
            pl.BlockSpec((H,), lambda i: (0,)),
        ],
        out_specs=[
            pl.BlockSpec((BLK, H), lambda i: (i, 0)),
            pl.BlockSpec((8, BLK), lambda i: (0, i)),
            pl.BlockSpec((8, BLK), lambda i: (0, i)),
            pl.BlockSpec((8, 128), lambda i: (0, 0)),
            pl.BlockSpec((8, 128), lambda i: (0, 0)),
        ],
        out_shape=[
            jax.ShapeDtypeStruct((NP, H), f32),
            jax.ShapeDtypeStruct((8, NP), f32),
            jax.ShapeDtypeStruct((8, NP), f32),
            jax.ShapeDtypeStruct((8, 128), f32),
            jax.ShapeDtypeStruct((8, 128), f32),
        ],
    )(x, w, al, ar)


def _merge_body(wp_ref, dp_ref, b_ref, h_ref):
    w = wp_ref[0] + wp_ref[1]
    dn = dp_ref[0] + dp_ref[1]
    h_ref[...] = _selu(w / (dn[:, None] + 1e-9) + b_ref[...][None, :])


def _merge(wparts, dparts, b):
    return pl.pallas_call(
        _merge_body,
        grid=(GRID,),
        in_specs=[
            pl.BlockSpec((2, BLK, H), lambda i: (0, i, 0)),
            pl.BlockSpec((2, BLK), lambda i: (0, i)),
            pl.BlockSpec((H,), lambda i: (0,)),
        ],
        out_specs=pl.BlockSpec((BLK, H), lambda i: (i, 0)),
        out_shape=jax.ShapeDtypeStruct((NP, H), jnp.float32),
    )(wparts, dparts, b)


def _merge_proj_body(wp_ref, dp_ref, b_ref, w_ref, al_ref, ar_ref,
                     h_ref, el_ref, er_ref, mel_ref, mer_ref):
    i = pl.program_id(0)
    wsum = wp_ref[0] + wp_ref[1]
    dn = dp_ref[0] + dp_ref[1]
    h2 = _selu(wsum / (dn[:, None] + 1e-9) + b_ref[...][None, :])
    h = jnp.dot(h2, w_ref[...], preferred_element_type=jnp.float32)
    h_ref[...] = h
    el = jnp.sum(h * al_ref[...][None, :], axis=1)
    er = jnp.sum(h * ar_ref[...][None, :], axis=1)
    el_ref[...] = jnp.broadcast_to(el[None, :], (8, BLK))
    er_ref[...] = jnp.broadcast_to(er[None, :], (8, BLK))
    ml = jnp.full((8, 128), jnp.max(el), dtype=jnp.float32)
    mr = jnp.full((8, 128), jnp.max(er), dtype=jnp.float32)

    @pl.when(i == 0)
    def _():
        mel_ref[...] = ml
        mer_ref[...] = mr

    @pl.when(i > 0)
    def _():
        mel_ref[...] = jnp.maximum(mel_ref[...], ml)
        mer_ref[...] = jnp.maximum(mer_ref[...], mr)


def _merge_project(wparts, dparts, b, w, al, ar):
    f32 = jnp.float32
    return pl.pallas_call(
        _merge_proj_body,
        grid=(GRID,),
        in_specs=[
            pl.BlockSpec((2, BLK, H), lambda i: (0, i, 0)),
            pl.BlockSpec((2, BLK), lambda i: (0, i)),
            pl.BlockSpec((H,), lambda i: (0,)),
            pl.BlockSpec((H, H), lambda i: (0, 0)),
            pl.BlockSpec((H,), lambda i: (0,)),
            pl.BlockSpec((H,), lambda i: (0,)),
        ],
        out_specs=[
            pl.BlockSpec((BLK, H), lambda i: (i, 0)),
            pl.BlockSpec((8, BLK), lambda i: (0, i)),
            pl.BlockSpec((8, BLK), lambda i: (0, i)),
            pl.BlockSpec((8, 128), lambda i: (0, 0)),
            pl.BlockSpec((8, 128), lambda i: (0, 0)),
        ],
        out_shape=[
            jax.ShapeDtypeStruct((NP, H), f32),
            jax.ShapeDtypeStruct((8, NP), f32),
            jax.ShapeDtypeStruct((8, NP), f32),
            jax.ShapeDtypeStruct((8, 128), f32),
            jax.ShapeDtypeStruct((8, 128), f32),
        ],
    )(wparts, dparts, b, w, al, ar)


def _head_body(wp_ref, dp_ref, b_ref, f1w_ref, f1b_ref, f2w_ref, f2b_ref,
               o_ref, gsum_ref):
    i = pl.program_id(0)
    w = wp_ref[0] + wp_ref[1]
    dn = dp_ref[0] + dp_ref[1]
    h3 = _selu(w / (dn[:, None] + 1e-9) + b_ref[...][None, :])
    row = lax.broadcasted_iota(jnp.int32, (BLK, H), 0) + i * BLK
    h3 = jnp.where(row < N, h3, 0.0)
    s = jnp.broadcast_to(jnp.sum(h3, axis=0)[None, :], (8, H))

    @pl.when(i == 0)
    def _():
        gsum_ref[...] = s

    @pl.when(i > 0)
    def _():
        gsum_ref[...] = gsum_ref[...] + s

    @pl.when(i == GRID - 1)
    def _():
        g = gsum_ref[0:1, :] * (1.0 / N)
        z = _selu(jnp.dot(g, f1w_ref[...], preferred_element_type=jnp.float32)
                  + f1b_ref[...][None, :])
        z = (jnp.dot(z, f2w_ref[...], preferred_element_type=jnp.float32)
             + f2b_ref[...][None, :])
        m = jnp.max(z, axis=1, keepdims=True)
        lse = m + jnp.log(jnp.sum(jnp.exp(z - m), axis=1, keepdims=True))
        o_ref[...] = z - lse


def _head(wparts, dparts, b, f1w, f1b, f2w, f2b):
    h2 = f1w.shape[1]
    c = f2w.shape[1]
    return pl.pallas_call(
        _head_body,
        grid=(GRID,),
        in_specs=[
            pl.BlockSpec((2, BLK, H), lambda i: (0, i, 0)),
            pl.BlockSpec((2, BLK), lambda i: (0, i)),
            pl.BlockSpec((H,), lambda i: (0,)),
            pl.BlockSpec((H, h2), lambda i: (0, 0)),
            pl.BlockSpec((h2,), lambda i: (0,)),
            pl.BlockSpec((h2, c), lambda i: (0, 0)),
            pl.BlockSpec((c,), lambda i: (0,)),
        ],
        out_specs=pl.BlockSpec((1, c), lambda i: (0, 0)),
        out_shape=jax.ShapeDtypeStruct((1, c), jnp.float32),
        scratch_shapes=[pltpu.VMEM((8, H), jnp.float32)],
    )(wparts, dparts, b, f1w, f1b, f2w, f2b)


def _edge_pass(h, el8, er8, mel, mer, edge_index):
    f32 = jnp.float32
    mesh = plsc.VectorSubcoreMesh(core_axis_name="c", subcore_axis_name="s")
    cp = pltpu.CompilerParams()
    if "needs_layout_passes" in pltpu.CompilerParams.__dataclass_fields__:
        cp = dataclasses.replace(cp, needs_layout_passes=False)
    if "use_tc_tiling_on_sc" in pltpu.CompilerParams.__dataclass_fields__:
        cp = dataclasses.replace(cp, use_tc_tiling_on_sc=False)

    @functools.partial(
        pl.kernel,
        compiler_params=cp,
        out_type=[
            jax.ShapeDtypeStruct((NC, NP, H), f32),
            jax.ShapeDtypeStruct((NC, NP), f32),
        ],
        mesh=mesh,
        scratch_types=(
            [
                pltpu.VMEM((NP,), f32),
                pltpu.VMEM((NP,), f32),
                pltpu.VMEM((16,), f32),
                pltpu.VMEM((16,), f32),
                pltpu.VMEM((NCH, CH), jnp.int32),
                pltpu.VMEM((NCH, CH), jnp.int32),
            ]
            + [pltpu.VMEM((CH, H), f32)] * 3
            + [pltpu.VMEM((CH,), f32)] * 3
            + [pltpu.SemaphoreType.DMA] * 9
            + [
                pltpu.VMEM_SHARED((NP, H), f32),
                pltpu.VMEM_SHARED((NP,), f32),
            ]
        ),
    )
    def k(h_hbm, el_hbm, er_hbm, mel_hbm, mer_hbm, ei_hbm, wout_hbm, dout_hbm,
          *scr):
        el_v, er_v, gm_v, gm2_v, src_a, dst_a = scr[0:6]
        rows, eeb = scr[6:9], scr[9:12]
        sg, sr, se = scr[12:15], scr[15:18], scr[18:21]
        w_sh, d_sh = scr[21], scr[22]

        c = lax.axis_index("c")
        s = lax.axis_index("s")
        wid = c * NS + s

        def issue_gather(ci, b):
            pltpu.async_copy(h_hbm.at[src_a.at[ci]], rows[b], sg[b])

        def wait_gather(b):
            pltpu.make_async_copy(h_hbm.at[src_a.at[0]], rows[b],
                                  sg[b]).wait()

        def issue_scatter(ci, b):
            pltpu.async_copy(rows[b], w_sh.at[dst_a.at[ci]], sr[b], add=True)
            pltpu.async_copy(eeb[b], d_sh.at[dst_a.at[ci]], se[b], add=True)

        def wait_scatter(b):
            pltpu.make_async_copy(rows[b], w_sh.at[dst_a.at[0]],
                                  sr[b]).wait()
            pltpu.make_async_copy(eeb[b], d_sh.at[dst_a.at[0]],
                                  se[b]).wait()

        def compute(ci, b, gmv):
            @plsc.parallel_loop(0, CH // 16, 1, unroll=5)
            def _grp(g):
                sv = src_a[ci, pl.ds(g * 16, 16)]
                dv = dst_a[ci, pl.ds(g * 16, 16)]
                e = plsc.load_gather(el_v, [sv]) + plsc.load_gather(er_v, [dv])
                e = jnp.where(e >= 0.0, e, 0.2 * e)
                eeb[b][pl.ds(g * 16, 16)] = jnp.exp(e - gmv)

            @plsc.parallel_loop(0, CH, 1, unroll=8)
            def _edge(j):
                wsplat = plsc.load_gather(
                    eeb[b], [jnp.full((16,), 0, jnp.int32) + j])
                for q in range(H // 16):
                    rows[b][j, pl.ds(q * 16, 16)] = (
                        rows[b][j, pl.ds(q * 16, 16)] * wsplat)

        pltpu.sync_copy(el_hbm.at[0], el_v)
        pltpu.sync_copy(er_hbm.at[0], er_v)
        pltpu.sync_copy(mel_hbm.at[0, pl.ds(0, 16)], gm_v)
        pltpu.sync_copy(mer_hbm.at[0, pl.ds(0, 16)], gm2_v)
        pltpu.sync_copy(ei_hbm.at[pl.ds(wid * NCH, NCH)], src_a)
        pltpu.sync_copy(ei_hbm.at[pl.ds((NC * NS + wid) * NCH, NCH)], dst_a)
        gmv = gm_v[...] + gm2_v[...]

        @plsc.parallel_loop(0, CH, 1, unroll=8)
        def _z(r):
            for q in range(H // 16):
                rows[0][r, pl.ds(q * 16, 16)] = jnp.zeros((16,), f32)

        for t in range(RPT // CH):
            pltpu.sync_copy(rows[0], w_sh.at[pl.ds(s * RPT + t * CH, CH)])
        for t in range(RPT // H):
            pltpu.sync_copy(rows[0].at[0], d_sh.at[pl.ds(s * RPT + t * H, H)])
        plsc.subcore_barrier()

        issue_gather(0, 0)

        def body(ci, i, r, last):
            b, b1 = r % 3, (r + 1) % 3
            if not last:
                if i is None:
                    wait_scatter(b1)
                else:
                    @pl.when(3 * i + r >= 2)
                    def _():
                        wait_scatter(b1)
                issue_gather(ci + 1, b1)
            wait_gather(b)
            compute(ci, b, gmv)
            issue_scatter(ci, b)

        @pl.loop(0, (NCH - 2) // 3)
        def _i(i):
            for r in range(3):
                body(3 * i + r, i, r, False)

        body(NCH - 2, None, (NCH - 2) % 3, False)
        body(NCH - 1, None, (NCH - 1) % 3, True)
        for b in range(3):
            wait_scatter(b)

        plsc.subcore_barrier()
        pltpu.sync_copy(w_sh.at[pl.ds(s * RPT, RPT)],
                        wout_hbm.at[c, pl.ds(s * RPT, RPT)])
        pltpu.sync_copy(d_sh.at[pl.ds(s * RPT, RPT)],
                        dout_hbm.at[c, pl.ds(s * RPT, RPT)])

    return k(h, el8, er8, mel, mer, edge_index.reshape(2 * E // CH, CH))


def kernel(x, edge_index, W1, al1, ar1, b1, W2, al2, ar2, b2,
           fc1_W, fc1_b, fc2_W, fc2_b):
    x = jnp.pad(x.astype(jnp.float32), ((0, NP - N), (0, 0)))

    h1, el1, er1, mel1, mer1 = _project(x, W1, al1, ar1)
    w1p, d1p = _edge_pass(h1, el1, er1, mel1, mer1, edge_index)
    h2p, el2, er2, mel2, mer2 = _merge_project(w1p, d1p, b1, W2, al2, ar2)
    w2p, d2p = _edge_pass(h2p, el2, er2, mel2, mer2, edge_index)
    return _head(w2p, d2p, b2, fc1_W, fc1_b, fc2_W, fc2_b)
